# baseline (speedup 1.0000x reference)
import dataclasses
import functools

import jax
import jax.numpy as jnp
from jax import lax
from jax.experimental import pallas as pl
from jax.experimental.pallas import tpu as pltpu
from jax.experimental.pallas import tpu_sc as plsc

BATCH = 16384
NC, NS = 2, 16
NW = NC * NS
B_PER_W = BATCH // NW

GMF_D = 16
MLP_D = 32

TC_BLOCK = 2048
TC_GRID = BATCH // TC_BLOCK


def _sc_gather(gmf_u_tab, gmf_i_tab, mlp_u_tab, mlp_i_tab, uidx, iidx):
    mesh = plsc.VectorSubcoreMesh(core_axis_name="c", subcore_axis_name="s")

    out_type = [
        jax.ShapeDtypeStruct((BATCH, GMF_D), jnp.float32),
        jax.ShapeDtypeStruct((BATCH, GMF_D), jnp.float32),
        jax.ShapeDtypeStruct((BATCH, MLP_D), jnp.float32),
        jax.ShapeDtypeStruct((BATCH, MLP_D), jnp.float32),
    ]
    scratch_types = [
        pltpu.VMEM((B_PER_W,), jnp.int32),
        pltpu.VMEM((B_PER_W,), jnp.int32),
        pltpu.SemaphoreType.DMA,
        pltpu.SemaphoreType.DMA,
        pltpu.SemaphoreType.DMA,
        pltpu.SemaphoreType.DMA,
    ]

    cp = pltpu.CompilerParams()
    if "needs_layout_passes" in pltpu.CompilerParams.__dataclass_fields__:
        cp = dataclasses.replace(cp, needs_layout_passes=False)

    @functools.partial(pl.kernel, mesh=mesh, out_type=out_type,
                       scratch_types=scratch_types, compiler_params=cp)
    def k(gu_hbm, gi_hbm, mu_hbm, mi_hbm, ui_hbm, ii_hbm,
          out_gu, out_gi, out_mu, out_mi,
          uvmem, ivmem, sem0, sem1, sem2, sem3):
        wid = lax.axis_index("s") * NC + lax.axis_index("c")
        base = wid * B_PER_W

        pltpu.sync_copy(ui_hbm.at[pl.ds(base, B_PER_W)], uvmem)
        pltpu.sync_copy(ii_hbm.at[pl.ds(base, B_PER_W)], ivmem)

        lanes = lax.iota(jnp.int32, 16)

        @pl.loop(0, B_PER_W // 16)
        def _(g):
            uvec = uvmem[pl.ds(g * 16, 16)]
            ivec = ivmem[pl.ds(g * 16, 16)]
            for j in range(16):
                iu = jnp.max(jnp.where(lanes == j, uvec, 0))
                ii = jnp.max(jnp.where(lanes == j, ivec, 0))
                b = base + g * 16 + j
                pltpu.make_async_copy(
                    gu_hbm.at[pl.ds(iu, 1)], out_gu.at[pl.ds(b, 1)],
                    sem0).start()
                pltpu.make_async_copy(
                    gi_hbm.at[pl.ds(ii, 1)], out_gi.at[pl.ds(b, 1)],
                    sem1).start()
                pltpu.make_async_copy(
                    mu_hbm.at[pl.ds(iu, 1)], out_mu.at[pl.ds(b, 1)],
                    sem2).start()
                pltpu.make_async_copy(
                    mi_hbm.at[pl.ds(ii, 1)], out_mi.at[pl.ds(b, 1)],
                    sem3).start()

        full = pl.ds(base, B_PER_W)
        pltpu.make_async_copy(gu_hbm.at[pl.ds(0, B_PER_W)],
                              out_gu.at[full], sem0).wait()
        pltpu.make_async_copy(gi_hbm.at[pl.ds(0, B_PER_W)],
                              out_gi.at[full], sem1).wait()
        pltpu.make_async_copy(mu_hbm.at[pl.ds(0, B_PER_W)],
                              out_mu.at[full], sem2).wait()
        pltpu.make_async_copy(mi_hbm.at[pl.ds(0, B_PER_W)],
                              out_mi.at[full], sem3).wait()

    return k(gmf_u_tab, gmf_i_tab, mlp_u_tab, mlp_i_tab, uidx, iidx)


def _tc_mlp_kernel(gu_ref, gi_ref, mu_ref, mi_ref,
                   w0_ref, b0_ref, w1_ref, b1_ref, w2_ref, b2_ref,
                   wg_ref, wm_ref, out_ref):
    xu = mu_ref[...]
    xi = mi_ref[...]
    w0a = w0_ref[0:MLP_D, :]
    w0b = w0_ref[MLP_D:2 * MLP_D, :]
    h = (jnp.dot(xu, w0a, preferred_element_type=jnp.float32)
         + jnp.dot(xi, w0b, preferred_element_type=jnp.float32)
         + b0_ref[...])
    h = jnp.maximum(h, 0.0)
    h = jnp.dot(h, w1_ref[...], preferred_element_type=jnp.float32) + b1_ref[...]
    h = jnp.maximum(h, 0.0)
    h = jnp.dot(h, w2_ref[...], preferred_element_type=jnp.float32) + b2_ref[...]
    h = jnp.maximum(h, 0.0)
    g = gu_ref[...] * gi_ref[...]
    s = jnp.sum(g * wg_ref[...], axis=-1) + jnp.sum(h * wm_ref[...], axis=-1)
    out_ref[0, 0, :] = jax.nn.sigmoid(s)


def kernel(user_ids, item_ids, gmf_user_emb, gmf_item_emb,
           mlp_user_emb, mlp_item_emb, W0, b0, W1, b1, W2, b2, Wout):
    uid = user_ids.astype(jnp.int32)
    iid = item_ids.astype(jnp.int32)

    gu, gi, mu, mi = _sc_gather(
        gmf_user_emb, gmf_item_emb, mlp_user_emb, mlp_item_emb, uid, iid)

    b0r = b0.reshape(1, -1)
    b1r = b1.reshape(1, -1)
    b2r = b2.reshape(1, -1)
    wg = Wout[:GMF_D, 0].reshape(1, GMF_D)
    wm = Wout[GMF_D:, 0].reshape(1, -1)

    full = lambda shape: pl.BlockSpec(shape, lambda i: (0,) * len(shape))
    out = pl.pallas_call(
        _tc_mlp_kernel,
        grid=(TC_GRID,),
        in_specs=[
            pl.BlockSpec((TC_BLOCK, GMF_D), lambda i: (i, 0)),
            pl.BlockSpec((TC_BLOCK, GMF_D), lambda i: (i, 0)),
            pl.BlockSpec((TC_BLOCK, MLP_D), lambda i: (i, 0)),
            pl.BlockSpec((TC_BLOCK, MLP_D), lambda i: (i, 0)),
            full(W0.shape), full(b0r.shape),
            full(W1.shape), full(b1r.shape),
            full(W2.shape), full(b2r.shape),
            full(wg.shape), full(wm.shape),
        ],
        out_specs=pl.BlockSpec((1, 1, TC_BLOCK), lambda i: (i, 0, 0)),
        out_shape=jax.ShapeDtypeStruct((TC_GRID, 1, TC_BLOCK), jnp.float32),
    )(gu, gi, mu, mi, W0, b0r, W1, b1r, W2, b2r, wg, wm)

    return out.reshape(BATCH)


# speedup vs baseline: 1.8550x; 1.8550x over previous
import dataclasses
import functools

import jax
import jax.numpy as jnp
from jax import lax
from jax.experimental import pallas as pl
from jax.experimental.pallas import tpu as pltpu
from jax.experimental.pallas import tpu_sc as plsc

BATCH = 16384
NC, NS = 2, 16
NW = NC * NS
B_PER_W = BATCH // NW

GMF_D = 16
MLP_D = 32

TC_BLOCK = 2048
TC_GRID = BATCH // TC_BLOCK


def _sc_gather(gmf_u_tab, gmf_i_tab, mlp_u_tab, mlp_i_tab, uidx, iidx):
    mesh = plsc.VectorSubcoreMesh(core_axis_name="c", subcore_axis_name="s")

    out_type = [
        jax.ShapeDtypeStruct((BATCH, GMF_D), jnp.float32),
        jax.ShapeDtypeStruct((BATCH, GMF_D), jnp.float32),
        jax.ShapeDtypeStruct((BATCH, MLP_D), jnp.float32),
        jax.ShapeDtypeStruct((BATCH, MLP_D), jnp.float32),
    ]
    scratch_types = [
        pltpu.VMEM((B_PER_W,), jnp.int32),
        pltpu.VMEM((B_PER_W,), jnp.int32),
        pltpu.VMEM((B_PER_W // 4, GMF_D), jnp.float32),
        pltpu.VMEM((B_PER_W // 4, GMF_D), jnp.float32),
        pltpu.VMEM((B_PER_W // 4, MLP_D), jnp.float32),
        pltpu.VMEM((B_PER_W // 4, MLP_D), jnp.float32),
        pltpu.SemaphoreType.DMA,
        pltpu.SemaphoreType.DMA,
        pltpu.SemaphoreType.DMA,
        pltpu.SemaphoreType.DMA,
    ]

    cp = pltpu.CompilerParams()
    if "needs_layout_passes" in pltpu.CompilerParams.__dataclass_fields__:
        cp = dataclasses.replace(cp, needs_layout_passes=False)

    @functools.partial(pl.kernel, mesh=mesh, out_type=out_type,
                       scratch_types=scratch_types, compiler_params=cp)
    def k(gu_hbm, gi_hbm, mu_hbm, mi_hbm, ui_hbm, ii_hbm,
          out_gu, out_gi, out_mu, out_mi,
          uvmem, ivmem, gu_v, gi_v, mu_v, mi_v, sem0, sem1, sem2, sem3):
        wid = lax.axis_index("s") * NC + lax.axis_index("c")
        base = wid * B_PER_W

        pltpu.sync_copy(ui_hbm.at[pl.ds(base, B_PER_W)], uvmem)
        pltpu.sync_copy(ii_hbm.at[pl.ds(base, B_PER_W)], ivmem)

        lanes = lax.iota(jnp.int32, 16)
        half = B_PER_W // 4

        for c in range(4):
            @pl.loop(0, half // 16)
            def _(g, c=c):
                uvec = uvmem[pl.ds(c * half + g * 16, 16)]
                ivec = ivmem[pl.ds(c * half + g * 16, 16)]
                for j in range(16):
                    iu = jnp.max(jnp.where(lanes == j, uvec, 0))
                    ii = jnp.max(jnp.where(lanes == j, ivec, 0))
                    b = g * 16 + j
                    pltpu.make_async_copy(
                        gu_hbm.at[pl.ds(iu, 1)], gu_v.at[pl.ds(b, 1)],
                        sem0).start()
                    pltpu.make_async_copy(
                        gi_hbm.at[pl.ds(ii, 1)], gi_v.at[pl.ds(b, 1)],
                        sem1).start()
                    pltpu.make_async_copy(
                        mu_hbm.at[pl.ds(iu, 1)], mu_v.at[pl.ds(b, 1)],
                        sem2).start()
                    pltpu.make_async_copy(
                        mi_hbm.at[pl.ds(ii, 1)], mi_v.at[pl.ds(b, 1)],
                        sem3).start()

            pltpu.make_async_copy(gu_hbm.at[pl.ds(0, half)], gu_v, sem0).wait()
            pltpu.make_async_copy(gi_hbm.at[pl.ds(0, half)], gi_v, sem1).wait()
            pltpu.make_async_copy(mu_hbm.at[pl.ds(0, half)], mu_v, sem2).wait()
            pltpu.make_async_copy(mi_hbm.at[pl.ds(0, half)], mi_v, sem3).wait()

            dst = pl.ds(base + c * half, half)
            pltpu.sync_copy(gu_v, out_gu.at[dst])
            pltpu.sync_copy(gi_v, out_gi.at[dst])
            pltpu.sync_copy(mu_v, out_mu.at[dst])
            pltpu.sync_copy(mi_v, out_mi.at[dst])

    return k(gmf_u_tab, gmf_i_tab, mlp_u_tab, mlp_i_tab, uidx, iidx)


def _tc_mlp_kernel(gu_ref, gi_ref, mu_ref, mi_ref,
                   w0_ref, b0_ref, w1_ref, b1_ref, w2_ref, b2_ref,
                   wg_ref, wm_ref, out_ref):
    xu = mu_ref[...]
    xi = mi_ref[...]
    w0a = w0_ref[0:MLP_D, :]
    w0b = w0_ref[MLP_D:2 * MLP_D, :]
    h = (jnp.dot(xu, w0a, preferred_element_type=jnp.float32)
         + jnp.dot(xi, w0b, preferred_element_type=jnp.float32)
         + b0_ref[...])
    h = jnp.maximum(h, 0.0)
    h = jnp.dot(h, w1_ref[...], preferred_element_type=jnp.float32) + b1_ref[...]
    h = jnp.maximum(h, 0.0)
    h = jnp.dot(h, w2_ref[...], preferred_element_type=jnp.float32) + b2_ref[...]
    h = jnp.maximum(h, 0.0)
    g = gu_ref[...] * gi_ref[...]
    s = jnp.sum(g * wg_ref[...], axis=-1) + jnp.sum(h * wm_ref[...], axis=-1)
    out_ref[0, 0, :] = jax.nn.sigmoid(s)


def kernel(user_ids, item_ids, gmf_user_emb, gmf_item_emb,
           mlp_user_emb, mlp_item_emb, W0, b0, W1, b1, W2, b2, Wout):
    uid = user_ids.astype(jnp.int32)
    iid = item_ids.astype(jnp.int32)

    gu, gi, mu, mi = _sc_gather(
        gmf_user_emb, gmf_item_emb, mlp_user_emb, mlp_item_emb, uid, iid)

    b0r = b0.reshape(1, -1)
    b1r = b1.reshape(1, -1)
    b2r = b2.reshape(1, -1)
    wg = Wout[:GMF_D, 0].reshape(1, GMF_D)
    wm = Wout[GMF_D:, 0].reshape(1, -1)

    full = lambda shape: pl.BlockSpec(shape, lambda i: (0,) * len(shape))
    out = pl.pallas_call(
        _tc_mlp_kernel,
        grid=(TC_GRID,),
        in_specs=[
            pl.BlockSpec((TC_BLOCK, GMF_D), lambda i: (i, 0)),
            pl.BlockSpec((TC_BLOCK, GMF_D), lambda i: (i, 0)),
            pl.BlockSpec((TC_BLOCK, MLP_D), lambda i: (i, 0)),
            pl.BlockSpec((TC_BLOCK, MLP_D), lambda i: (i, 0)),
            full(W0.shape), full(b0r.shape),
            full(W1.shape), full(b1r.shape),
            full(W2.shape), full(b2r.shape),
            full(wg.shape), full(wm.shape),
        ],
        out_specs=pl.BlockSpec((1, 1, TC_BLOCK), lambda i: (i, 0, 0)),
        out_shape=jax.ShapeDtypeStruct((TC_GRID, 1, TC_BLOCK), jnp.float32),
    )(gu, gi, mu, mi, W0, b0r, W1, b1r, W2, b2r, wg, wm)

    return out.reshape(BATCH)
